# baseline (speedup 1.0000x reference)
import functools

import jax
import jax.numpy as jnp
from jax import lax
from jax.experimental import pallas as pl
from jax.experimental.pallas import tpu as pltpu
from jax.experimental.pallas import tpu_sc as plsc

_N = 10000
_E = 320000
_D = 128

_NC = 2
_NS = 16
_NW = _NC * _NS
_EPW = _E // _NW
_K = 80
_NFULL = _EPW // _K
_KT = _K
_RQ = 624
_TAIL = _N - _NS * _RQ
_DEGW = 16

_mesh = plsc.VectorSubcoreMesh(core_axis_name="c", subcore_axis_name="s")
_sc_params = pltpu.CompilerParams(needs_layout_passes=False)


_ECH = 2000


@functools.partial(
    pl.kernel,
    mesh=_mesh,
    out_type=jax.ShapeDtypeStruct((_NW, 1, _N), jnp.float32),
    scratch_types=[
        pltpu.VMEM((_ECH,), jnp.int32),
        pltpu.VMEM((1, _N), jnp.float32),
    ],
    compiler_params=_sc_params,
)
def _deg_sc(dst_hbm, out_hbm, idx_v, deg_v):
    c = lax.axis_index("c")
    s = lax.axis_index("s")
    wid = s * _NC + c

    @pl.loop(0, _N, step=16)
    def _(i):
        deg_v[0, pl.ds(i, 16)] = jnp.zeros((16,), jnp.float32)

    row0 = jnp.zeros((16,), jnp.int32)
    ones = jnp.ones((16,), jnp.float32)

    @pl.loop(0, _EPW, step=_ECH)
    def _(eb):
        pltpu.sync_copy(dst_hbm.at[pl.ds(wid * _EPW + eb, _ECH)], idx_v)

        @pl.loop(0, _ECH, step=16)
        def _(j):
            idx16 = idx_v[pl.ds(j, 16)]
            plsc.addupdate_scatter(deg_v, [row0, idx16], ones)

    pltpu.sync_copy(deg_v, out_hbm.at[wid])


_NBUF = 4
_TOUT = _NFULL // _NBUF


@functools.partial(
    pl.kernel,
    mesh=_mesh,
    out_type=jax.ShapeDtypeStruct((_NC, _N, _D), jnp.float32),
    scratch_types=[
        pltpu.VMEM((_NBUF, _K), jnp.int32),
        pltpu.VMEM((_NBUF, _K), jnp.int32),
        pltpu.VMEM((_KT,), jnp.int32),
        pltpu.VMEM((_KT,), jnp.int32),
        pltpu.VMEM((_NBUF, _K, _D), jnp.float32),
        pltpu.VMEM_SHARED((_N, _D), jnp.float32),
        pltpu.SemaphoreType.DMA((_NBUF,)),
        pltpu.SemaphoreType.DMA((_NBUF,)),
        pltpu.SemaphoreType.DMA((_NBUF,)),
        pltpu.SemaphoreType.DMA((_NBUF,)),
    ],
)
def _edge_sc(h_hbm, src_hbm, dst_hbm, out_hbm, src_v, dst_v, srct_v, dstt_v,
             rows_v, acc_sh, sem_is, sem_id, sem_g, sem_sc):
    c = lax.axis_index("c")
    s = lax.axis_index("s")
    wid = s * _NC + c
    ebase = wid * _EPW

    pltpu.sync_copy(h_hbm.at[pl.ds(s * _RQ, _RQ)],
                    acc_sh.at[pl.ds(s * _RQ, _RQ)])

    @pl.when(s == _NS - 1)
    def _():
        pltpu.sync_copy(h_hbm.at[pl.ds(_NS * _RQ, _TAIL)],
                        acc_sh.at[pl.ds(_NS * _RQ, _TAIL)])

    plsc.subcore_barrier()

    for b in range(_NBUF):
        pltpu.async_copy(src_hbm.at[pl.ds(ebase + b * _K, _K)],
                         src_v.at[b], sem_is.at[b])
        pltpu.async_copy(dst_hbm.at[pl.ds(ebase + b * _K, _K)],
                         dst_v.at[b], sem_id.at[b])

    @pl.loop(0, _TOUT)
    def _(t):
        for b in range(_NBUF):
            pltpu.make_async_copy(src_hbm.at[pl.ds(0, _K)],
                                  src_v.at[b], sem_is.at[b]).wait()
            pltpu.async_copy(h_hbm.at[src_v.at[b]], rows_v.at[b], sem_g.at[b])

        for b in range(_NBUF):
            pltpu.make_async_copy(h_hbm.at[src_v.at[b]],
                                  rows_v.at[b], sem_g.at[b]).wait()
            pltpu.make_async_copy(src_hbm.at[pl.ds(0, _K)],
                                  dst_v.at[b], sem_id.at[b]).wait()
            pltpu.async_copy(rows_v.at[b], acc_sh.at[dst_v.at[b]],
                             sem_sc.at[b], add=True)

        for b in range(_NBUF):
            pltpu.make_async_copy(rows_v.at[b], acc_sh.at[pl.ds(0, _K)],
                                  sem_sc.at[b]).wait()

            @pl.when(t < _TOUT - 1)
            def _():
                nxt = ebase + (t + 1) * _NBUF * _K + b * _K
                pltpu.async_copy(src_hbm.at[pl.ds(nxt, _K)],
                                 src_v.at[b], sem_is.at[b])
                pltpu.async_copy(dst_hbm.at[pl.ds(nxt, _K)],
                                 dst_v.at[b], sem_id.at[b])

    tbase = ebase + _NBUF * _TOUT * _K
    pltpu.async_copy(src_hbm.at[pl.ds(tbase, _KT)], srct_v, sem_is.at[0])
    pltpu.async_copy(dst_hbm.at[pl.ds(tbase, _KT)], dstt_v, sem_id.at[0])
    trows = rows_v.at[0].at[pl.ds(0, _KT)]
    pltpu.make_async_copy(src_hbm.at[pl.ds(0, _KT)],
                          srct_v, sem_is.at[0]).wait()
    pltpu.async_copy(h_hbm.at[srct_v], trows, sem_g.at[0])
    pltpu.make_async_copy(h_hbm.at[srct_v], trows, sem_g.at[0]).wait()
    pltpu.make_async_copy(src_hbm.at[pl.ds(0, _KT)],
                          dstt_v, sem_id.at[0]).wait()
    pltpu.sync_copy(trows, acc_sh.at[dstt_v], add=True)

    plsc.subcore_barrier()
    pltpu.sync_copy(acc_sh.at[pl.ds(s * _RQ, _RQ)],
                    out_hbm.at[c, pl.ds(s * _RQ, _RQ)])

    @pl.when(s == _NS - 1)
    def _():
        pltpu.sync_copy(acc_sh.at[pl.ds(_NS * _RQ, _TAIL)],
                        out_hbm.at[c, pl.ds(_NS * _RQ, _TAIL)])


def _mm0_body(x_ref, w0_ref, g_ref):
    g_ref[...] = jnp.dot(x_ref[...], w0_ref[...],
                         preferred_element_type=jnp.float32)


_mm0_tc = pl.pallas_call(
    _mm0_body,
    out_shape=jax.ShapeDtypeStruct((_N, _D), jnp.float32),
)


def _prep_body(degt_ref, g_ref, isq_ref, h0_ref):
    deg = jnp.sum(degt_ref[...], axis=1, keepdims=True) + 1.0
    isq = lax.rsqrt(deg)
    isq_ref[...] = isq
    h0_ref[...] = g_ref[...] * isq


_prep_tc = pl.pallas_call(
    _prep_body,
    out_shape=(
        jax.ShapeDtypeStruct((_N, 1), jnp.float32),
        jax.ShapeDtypeStruct((_N, _D), jnp.float32),
    ),
)


def _mid_body(p_ref, h0_ref, isq_ref, b0_ref, w1_ref, h1_ref):
    isq = isq_ref[...]
    agg = p_ref[0] + p_ref[1] - h0_ref[...]
    h = jax.nn.relu(agg * isq + b0_ref[...])
    h1 = jnp.dot(h, w1_ref[...], preferred_element_type=jnp.float32)
    h1_ref[...] = h1 * isq


_mid_tc = pl.pallas_call(
    _mid_body,
    out_shape=jax.ShapeDtypeStruct((_N, _D), jnp.float32),
)


def _final_body(q_ref, h1_ref, isq_ref, b1_ref, out_ref):
    agg = q_ref[0] + q_ref[1] - h1_ref[...]
    out_ref[...] = agg * isq_ref[...] + b1_ref[...]


_final_tc = pl.pallas_call(
    _final_body,
    out_shape=jax.ShapeDtypeStruct((_N, _D), jnp.float32),
)


def kernel(x, edge_index, W0, b0, W1, b1):
    src = edge_index[0]
    dst = edge_index[1]
    g = _mm0_tc(x, W0)
    deg_p = _deg_sc(dst)
    degt = jnp.transpose(deg_p[:, 0, :])
    isq, h0 = _prep_tc(degt, g)
    p = _edge_sc(h0, src, dst)
    h1 = _mid_tc(p, h0, isq, b0, W1)
    q = _edge_sc(h1, src, dst)
    return _final_tc(q, h1, isq, b1)


# speedup vs baseline: 1.0031x; 1.0031x over previous
import functools

import jax
import jax.numpy as jnp
from jax import lax
from jax.experimental import pallas as pl
from jax.experimental.pallas import tpu as pltpu
from jax.experimental.pallas import tpu_sc as plsc

_N = 10000
_E = 320000
_D = 128

_NC = 2
_NS = 16
_NW = _NC * _NS
_EPW = _E // _NW
_K = 80
_NFULL = _EPW // _K
_KT = _K
_RQ = 624
_TAIL = _N - _NS * _RQ
_DEGW = 16

_mesh = plsc.VectorSubcoreMesh(core_axis_name="c", subcore_axis_name="s")
_sc_params = pltpu.CompilerParams(needs_layout_passes=False)


_ECH = 2000


@functools.partial(
    pl.kernel,
    mesh=_mesh,
    out_type=jax.ShapeDtypeStruct((_NW, 1, _N), jnp.float32),
    scratch_types=[
        pltpu.VMEM((_ECH,), jnp.int32),
        pltpu.VMEM((1, _N), jnp.float32),
    ],
    compiler_params=_sc_params,
)
def _deg_sc(dst_hbm, out_hbm, idx_v, deg_v):
    c = lax.axis_index("c")
    s = lax.axis_index("s")
    wid = s * _NC + c

    @pl.loop(0, _N, step=16)
    def _(i):
        deg_v[0, pl.ds(i, 16)] = jnp.zeros((16,), jnp.float32)

    row0 = jnp.zeros((16,), jnp.int32)
    ones = jnp.ones((16,), jnp.float32)

    @pl.loop(0, _EPW, step=_ECH)
    def _(eb):
        pltpu.sync_copy(dst_hbm.at[pl.ds(wid * _EPW + eb, _ECH)], idx_v)

        @pl.loop(0, _ECH, step=16)
        def _(j):
            idx16 = idx_v[pl.ds(j, 16)]
            plsc.addupdate_scatter(deg_v, [row0, idx16], ones)

    pltpu.sync_copy(deg_v, out_hbm.at[wid])


_NBUF = 4
_TOUT = _NFULL // _NBUF


@functools.partial(
    pl.kernel,
    mesh=_mesh,
    out_type=jax.ShapeDtypeStruct((_NC, _N, _D), jnp.float32),
    scratch_types=[
        pltpu.VMEM((_NBUF, _K), jnp.int32),
        pltpu.VMEM((_NBUF, _K), jnp.int32),
        pltpu.VMEM((_KT,), jnp.int32),
        pltpu.VMEM((_KT,), jnp.int32),
        pltpu.VMEM((_NBUF, _K, _D), jnp.float32),
        pltpu.VMEM_SHARED((_N, _D), jnp.float32),
        pltpu.SemaphoreType.DMA((_NBUF,)),
        pltpu.SemaphoreType.DMA((_NBUF,)),
        pltpu.SemaphoreType.DMA((_NBUF,)),
        pltpu.SemaphoreType.DMA((_NBUF,)),
    ],
)
def _edge_sc(h_hbm, src_hbm, dst_hbm, out_hbm, src_v, dst_v, srct_v, dstt_v,
             rows_v, acc_sh, sem_is, sem_id, sem_g, sem_sc):
    c = lax.axis_index("c")
    s = lax.axis_index("s")
    wid = s * _NC + c
    ebase = wid * _EPW

    pltpu.sync_copy(h_hbm.at[pl.ds(s * _RQ, _RQ)],
                    acc_sh.at[pl.ds(s * _RQ, _RQ)])

    @pl.when(s == _NS - 1)
    def _():
        pltpu.sync_copy(h_hbm.at[pl.ds(_NS * _RQ, _TAIL)],
                        acc_sh.at[pl.ds(_NS * _RQ, _TAIL)])

    plsc.subcore_barrier()

    for b in range(_NBUF):
        pltpu.async_copy(src_hbm.at[pl.ds(ebase + b * _K, _K)],
                         src_v.at[b], sem_is.at[b])
        pltpu.async_copy(dst_hbm.at[pl.ds(ebase + b * _K, _K)],
                         dst_v.at[b], sem_id.at[b])

    @pl.loop(0, _TOUT)
    def _(t):
        for b in range(_NBUF):
            pltpu.make_async_copy(src_hbm.at[pl.ds(0, _K)],
                                  src_v.at[b], sem_is.at[b]).wait()
            pltpu.async_copy(h_hbm.at[src_v.at[b]], rows_v.at[b], sem_g.at[b])

        for b in range(_NBUF):
            pltpu.make_async_copy(h_hbm.at[src_v.at[b]],
                                  rows_v.at[b], sem_g.at[b]).wait()
            pltpu.make_async_copy(src_hbm.at[pl.ds(0, _K)],
                                  dst_v.at[b], sem_id.at[b]).wait()
            pltpu.async_copy(rows_v.at[b], acc_sh.at[dst_v.at[b]],
                             sem_sc.at[b], add=True)

        for b in range(_NBUF):
            pltpu.make_async_copy(rows_v.at[b], acc_sh.at[pl.ds(0, _K)],
                                  sem_sc.at[b]).wait()

            @pl.when(t < _TOUT - 1)
            def _():
                nxt = ebase + (t + 1) * _NBUF * _K + b * _K
                pltpu.async_copy(src_hbm.at[pl.ds(nxt, _K)],
                                 src_v.at[b], sem_is.at[b])
                pltpu.async_copy(dst_hbm.at[pl.ds(nxt, _K)],
                                 dst_v.at[b], sem_id.at[b])

    tbase = ebase + _NBUF * _TOUT * _K
    pltpu.async_copy(src_hbm.at[pl.ds(tbase, _KT)], srct_v, sem_is.at[0])
    pltpu.async_copy(dst_hbm.at[pl.ds(tbase, _KT)], dstt_v, sem_id.at[0])
    trows = rows_v.at[0].at[pl.ds(0, _KT)]
    pltpu.make_async_copy(src_hbm.at[pl.ds(0, _KT)],
                          srct_v, sem_is.at[0]).wait()
    pltpu.async_copy(h_hbm.at[srct_v], trows, sem_g.at[0])
    pltpu.make_async_copy(h_hbm.at[srct_v], trows, sem_g.at[0]).wait()
    pltpu.make_async_copy(src_hbm.at[pl.ds(0, _KT)],
                          dstt_v, sem_id.at[0]).wait()
    pltpu.sync_copy(trows, acc_sh.at[dstt_v], add=True)

    plsc.subcore_barrier()
    pltpu.sync_copy(acc_sh.at[pl.ds(s * _RQ, _RQ)],
                    out_hbm.at[c, pl.ds(s * _RQ, _RQ)])

    @pl.when(s == _NS - 1)
    def _():
        pltpu.sync_copy(acc_sh.at[pl.ds(_NS * _RQ, _TAIL)],
                        out_hbm.at[c, pl.ds(_NS * _RQ, _TAIL)])


def _prep_body(degt_ref, x_ref, w0_ref, isq_ref, h0_ref):
    deg = jnp.sum(degt_ref[...], axis=1, keepdims=True) + 1.0
    isq = lax.rsqrt(deg)
    isq_ref[...] = isq
    h0 = jnp.dot(x_ref[...], w0_ref[...], preferred_element_type=jnp.float32)
    h0_ref[...] = h0 * isq


_prep_tc = pl.pallas_call(
    _prep_body,
    out_shape=(
        jax.ShapeDtypeStruct((_N, 1), jnp.float32),
        jax.ShapeDtypeStruct((_N, _D), jnp.float32),
    ),
)


def _mid_body(p_ref, h0_ref, isq_ref, b0_ref, w1_ref, h1_ref):
    isq = isq_ref[...]
    agg = p_ref[0] + p_ref[1] - h0_ref[...]
    h = jax.nn.relu(agg * isq + b0_ref[...])
    h1 = jnp.dot(h, w1_ref[...], preferred_element_type=jnp.float32)
    h1_ref[...] = h1 * isq


_mid_tc = pl.pallas_call(
    _mid_body,
    out_shape=jax.ShapeDtypeStruct((_N, _D), jnp.float32),
)


def _final_body(q_ref, h1_ref, isq_ref, b1_ref, out_ref):
    agg = q_ref[0] + q_ref[1] - h1_ref[...]
    out_ref[...] = agg * isq_ref[...] + b1_ref[...]


_final_tc = pl.pallas_call(
    _final_body,
    out_shape=jax.ShapeDtypeStruct((_N, _D), jnp.float32),
)


def kernel(x, edge_index, W0, b0, W1, b1):
    src = edge_index[0]
    dst = edge_index[1]
    deg_p = _deg_sc(dst)
    degt = jnp.transpose(deg_p[:, 0, :])
    isq, h0 = _prep_tc(degt, x, W0)
    p = _edge_sc(h0, src, dst)
    h1 = _mid_tc(p, h0, isq, b0, W1)
    q = _edge_sc(h1, src, dst)
    return _final_tc(q, h1, isq, b1)


# speedup vs baseline: 1.0042x; 1.0011x over previous
import functools

import jax
import jax.numpy as jnp
from jax import lax
from jax.experimental import pallas as pl
from jax.experimental.pallas import tpu as pltpu
from jax.experimental.pallas import tpu_sc as plsc

_N = 10000
_E = 320000
_D = 128

_NC = 2
_NS = 16
_NW = _NC * _NS
_EPW = _E // _NW
_K = 80
_NFULL = _EPW // _K
_KT = _K
_RQ = 624
_TAIL = _N - _NS * _RQ

_mesh = plsc.VectorSubcoreMesh(core_axis_name="c", subcore_axis_name="s")
_sc_params = pltpu.CompilerParams(needs_layout_passes=False)


_ECH = 2000


@functools.partial(
    pl.kernel,
    mesh=_mesh,
    out_type=jax.ShapeDtypeStruct((_NW, 1, _N), jnp.float32),
    scratch_types=[
        pltpu.VMEM((_ECH,), jnp.int32),
        pltpu.VMEM((1, _N), jnp.float32),
    ],
    compiler_params=_sc_params,
)
def _deg_sc(dst_hbm, out_hbm, idx_v, deg_v):
    c = lax.axis_index("c")
    s = lax.axis_index("s")
    wid = s * _NC + c

    @pl.loop(0, _N, step=16)
    def _(i):
        deg_v[0, pl.ds(i, 16)] = jnp.zeros((16,), jnp.float32)

    row0 = jnp.zeros((16,), jnp.int32)
    ones = jnp.ones((16,), jnp.float32)

    @pl.loop(0, _EPW, step=_ECH)
    def _(eb):
        pltpu.sync_copy(dst_hbm.at[pl.ds(wid * _EPW + eb, _ECH)], idx_v)

        @pl.loop(0, _ECH, step=16)
        def _(j):
            idx16 = idx_v[pl.ds(j, 16)]
            plsc.addupdate_scatter(deg_v, [row0, idx16], ones)

    pltpu.sync_copy(deg_v, out_hbm.at[wid])


_NBUF = 4
_TOUT = _NFULL // _NBUF


@functools.partial(
    pl.kernel,
    mesh=_mesh,
    out_type=jax.ShapeDtypeStruct((_NC, _N, _D), jnp.float32),
    scratch_types=[
        pltpu.VMEM((_NBUF, _K), jnp.int32),
        pltpu.VMEM((_NBUF, _K), jnp.int32),
        pltpu.VMEM((_KT,), jnp.int32),
        pltpu.VMEM((_KT,), jnp.int32),
        pltpu.VMEM((_NBUF, _K, _D), jnp.float32),
        pltpu.VMEM_SHARED((_N, _D), jnp.float32),
        pltpu.SemaphoreType.DMA((_NBUF,)),
        pltpu.SemaphoreType.DMA((_NBUF,)),
        pltpu.SemaphoreType.DMA((_NBUF,)),
        pltpu.SemaphoreType.DMA((_NBUF,)),
    ],
)
def _edge_sc(h_hbm, src_hbm, dst_hbm, out_hbm, src_v, dst_v, srct_v, dstt_v,
             rows_v, acc_sh, sem_is, sem_id, sem_g, sem_sc):
    c = lax.axis_index("c")
    s = lax.axis_index("s")
    wid = s * _NC + c
    ebase = wid * _EPW

    pltpu.sync_copy(h_hbm.at[pl.ds(s * _RQ, _RQ)],
                    acc_sh.at[pl.ds(s * _RQ, _RQ)])

    @pl.when(s == _NS - 1)
    def _():
        pltpu.sync_copy(h_hbm.at[pl.ds(_NS * _RQ, _TAIL)],
                        acc_sh.at[pl.ds(_NS * _RQ, _TAIL)])

    plsc.subcore_barrier()

    for b in range(_NBUF):
        pltpu.async_copy(src_hbm.at[pl.ds(ebase + b * _K, _K)],
                         src_v.at[b], sem_is.at[b])
        pltpu.async_copy(dst_hbm.at[pl.ds(ebase + b * _K, _K)],
                         dst_v.at[b], sem_id.at[b])

    @pl.loop(0, _TOUT)
    def _(t):
        for b in range(_NBUF):
            pltpu.make_async_copy(src_hbm.at[pl.ds(0, _K)],
                                  src_v.at[b], sem_is.at[b]).wait()
            pltpu.async_copy(h_hbm.at[src_v.at[b]], rows_v.at[b], sem_g.at[b])

        for b in range(_NBUF):
            pltpu.make_async_copy(h_hbm.at[src_v.at[b]],
                                  rows_v.at[b], sem_g.at[b]).wait()
            pltpu.make_async_copy(src_hbm.at[pl.ds(0, _K)],
                                  dst_v.at[b], sem_id.at[b]).wait()
            pltpu.async_copy(rows_v.at[b], acc_sh.at[dst_v.at[b]],
                             sem_sc.at[b], add=True)

        for b in range(_NBUF):
            pltpu.make_async_copy(rows_v.at[b], acc_sh.at[pl.ds(0, _K)],
                                  sem_sc.at[b]).wait()

            @pl.when(t < _TOUT - 1)
            def _():
                nxt = ebase + (t + 1) * _NBUF * _K + b * _K
                pltpu.async_copy(src_hbm.at[pl.ds(nxt, _K)],
                                 src_v.at[b], sem_is.at[b])
                pltpu.async_copy(dst_hbm.at[pl.ds(nxt, _K)],
                                 dst_v.at[b], sem_id.at[b])

    tbase = ebase + _NBUF * _TOUT * _K
    pltpu.async_copy(src_hbm.at[pl.ds(tbase, _KT)], srct_v, sem_is.at[0])
    pltpu.async_copy(dst_hbm.at[pl.ds(tbase, _KT)], dstt_v, sem_id.at[0])
    trows = rows_v.at[0].at[pl.ds(0, _KT)]
    pltpu.make_async_copy(src_hbm.at[pl.ds(0, _KT)],
                          srct_v, sem_is.at[0]).wait()
    pltpu.async_copy(h_hbm.at[srct_v], trows, sem_g.at[0])
    pltpu.make_async_copy(h_hbm.at[srct_v], trows, sem_g.at[0]).wait()
    pltpu.make_async_copy(src_hbm.at[pl.ds(0, _KT)],
                          dstt_v, sem_id.at[0]).wait()
    pltpu.sync_copy(trows, acc_sh.at[dstt_v], add=True)

    plsc.subcore_barrier()
    pltpu.sync_copy(acc_sh.at[pl.ds(s * _RQ, _RQ)],
                    out_hbm.at[c, pl.ds(s * _RQ, _RQ)])

    @pl.when(s == _NS - 1)
    def _():
        pltpu.sync_copy(acc_sh.at[pl.ds(_NS * _RQ, _TAIL)],
                        out_hbm.at[c, pl.ds(_NS * _RQ, _TAIL)])


def _prep_body(degt_ref, x_ref, w0_ref, isq_ref, h0_ref):
    deg = jnp.sum(degt_ref[...], axis=1, keepdims=True) + 1.0
    isq = lax.rsqrt(deg)
    isq_ref[...] = isq
    h0 = jnp.dot(x_ref[...], w0_ref[...], preferred_element_type=jnp.float32)
    h0_ref[...] = h0 * isq


_prep_tc = pl.pallas_call(
    _prep_body,
    out_shape=(
        jax.ShapeDtypeStruct((_N, 1), jnp.float32),
        jax.ShapeDtypeStruct((_N, _D), jnp.float32),
    ),
)


def _mid_body(p_ref, h0_ref, isq_ref, b0_ref, w1_ref, h1_ref):
    isq = isq_ref[...]
    agg = p_ref[0] + p_ref[1] - h0_ref[...]
    h = jax.nn.relu(agg * isq + b0_ref[...])
    h1 = jnp.dot(h, w1_ref[...], preferred_element_type=jnp.float32)
    h1_ref[...] = h1 * isq


_mid_tc = pl.pallas_call(
    _mid_body,
    out_shape=jax.ShapeDtypeStruct((_N, _D), jnp.float32),
)


def _final_body(q_ref, h1_ref, isq_ref, b1_ref, out_ref):
    agg = q_ref[0] + q_ref[1] - h1_ref[...]
    out_ref[...] = agg * isq_ref[...] + b1_ref[...]


_final_tc = pl.pallas_call(
    _final_body,
    out_shape=jax.ShapeDtypeStruct((_N, _D), jnp.float32),
)


def kernel(x, edge_index, W0, b0, W1, b1):
    src = edge_index[0]
    dst = edge_index[1]
    deg_p = _deg_sc(dst)
    degt = jnp.transpose(deg_p[:, 0, :])
    isq, h0 = _prep_tc(degt, x, W0)
    p = _edge_sc(h0, src, dst)
    h1 = _mid_tc(p, h0, isq, b0, W1)
    q = _edge_sc(h1, src, dst)
    return _final_tc(q, h1, isq, b1)
